# Initial kernel scaffold; baseline (speedup 1.0000x reference)
import jax
import jax.numpy as jnp
from jax.experimental import pallas as pl


def kernel(queries, keys, values):
    raise NotImplementedError("write your pallas kernel here")


# speedup vs baseline: 2.7614x; 2.7614x over previous
import functools
import math

import jax
import jax.numpy as jnp
import numpy as np
from jax.experimental import pallas as pl
from jax.experimental.pallas import tpu as pltpu

_FACTOR = 5
_L = 2048
_U = 40
_NEG = -1e9


def _build_counts_t():
    idx_key = jax.random.key(42)
    idx = np.asarray(
        jax.random.randint(idx_key, (_L, _U), 0, _L), dtype=np.int64
    )
    counts_t = np.zeros((_L, _L), dtype=np.int8)
    np.add.at(counts_t, (idx.reshape(-1), np.repeat(np.arange(_L), _U)), 1)
    return counts_t


_COUNTS_T = _build_counts_t()


def _head_kernel(q_ref, k_ref, v_ref, c_ref, o_ref, m_ref, sel_ref, *, bq1, bq2):
    L = _L
    K = k_ref[0]
    V = v_ref[0]
    D = v_ref.shape[-1]
    scale = 1.0 / math.sqrt(D)

    def s1(i, _):
        qb = q_ref[0, pl.ds(i * bq1, bq1), :]
        st = jax.lax.dot_general(
            K, qb, (((1,), (1,)), ((), ())),
            preferred_element_type=jnp.float32,
        )
        c = c_ref[:, pl.ds(i * bq1, bq1)]
        cf = c.astype(jnp.float32)
        mx = jnp.max(jnp.where(cf > 0.5, st, -3e38), axis=0, keepdims=True)
        sm = jnp.sum(cf * st, axis=0, keepdims=True)
        m_ref[:, pl.ds(i * bq1, bq1)] = mx - sm * (1.0 / L)
        return 0

    jax.lax.fori_loop(0, L // bq1, s1, 0, unroll=False)

    M = m_ref[...]
    lane_iota = jax.lax.broadcasted_iota(jnp.int32, (1, L), 1)
    col_iota = jax.lax.broadcasted_iota(jnp.int32, (L, 1), 0)

    def tk(_, carry):
        m, sel = carry
        vmax = jnp.max(m)
        i = jnp.min(jnp.where(m == vmax, lane_iota, L))
        sel = jnp.where(col_iota == i, 1.0, sel)
        m = jnp.where(lane_iota == i, -3e38, m)
        return m, sel

    _, sel_col = jax.lax.fori_loop(
        0, _U, tk, (M, jnp.zeros((L, 1), jnp.float32)), unroll=False
    )
    sel_ref[...] = sel_col

    r2 = jax.lax.broadcasted_iota(jnp.int32, (bq2, bq2), 0)
    c2 = jax.lax.broadcasted_iota(jnp.int32, (bq2, bq2), 1)
    tri = (r2 >= c2).astype(jnp.float32)

    k_iota = jax.lax.broadcasted_iota(jnp.int32, (L, bq2), 0)
    q_iota = jax.lax.broadcasted_iota(jnp.int32, (L, bq2), 1)

    def s3(i, carry_sum):
        qb = q_ref[0, pl.ds(i * bq2, bq2), :]
        st = jax.lax.dot_general(
            K, qb, (((1,), (1,)), ((), ())),
            preferred_element_type=jnp.float32,
        ) * scale
        st = jnp.where(k_iota > q_iota + i * bq2, _NEG, st)
        st = st - jnp.max(st, axis=0, keepdims=True)
        e = jnp.exp(st)
        p = e / jnp.sum(e, axis=0, keepdims=True)
        attn = jax.lax.dot_general(
            p, V, (((0,), (0,)), ((), ())),
            preferred_element_type=jnp.float32,
        )

        vb = v_ref[0, pl.ds(i * bq2, bq2), :]
        ctx = jax.lax.dot_general(
            tri, vb, (((1,), (0,)), ((), ())),
            preferred_element_type=jnp.float32,
        ) + carry_sum

        selb = sel_ref[pl.ds(i * bq2, bq2), :]
        o_ref[0, pl.ds(i * bq2, bq2), :] = selb * attn + (1.0 - selb) * ctx
        return carry_sum + jnp.sum(vb, axis=0, keepdims=True)

    jax.lax.fori_loop(0, L // bq2, s3, jnp.zeros((1, D), jnp.float32),
                      unroll=False)


@jax.jit
def kernel(queries, keys, values):
    B, L, H, D = queries.shape
    Q = jnp.transpose(queries, (0, 2, 1, 3)).reshape(B * H, L, D)
    K = jnp.transpose(keys, (0, 2, 1, 3)).reshape(B * H, L, D)
    V = jnp.transpose(values, (0, 2, 1, 3)).reshape(B * H, L, D)
    counts_t = jnp.asarray(_COUNTS_T)

    bq1, bq2 = 256, 256
    out = pl.pallas_call(
        functools.partial(_head_kernel, bq1=bq1, bq2=bq2),
        grid=(B * H,),
        in_specs=[
            pl.BlockSpec((1, L, D), lambda h: (h, 0, 0)),
            pl.BlockSpec((1, L, D), lambda h: (h, 0, 0)),
            pl.BlockSpec((1, L, D), lambda h: (h, 0, 0)),
            pl.BlockSpec((L, L), lambda h: (0, 0)),
        ],
        out_specs=pl.BlockSpec((1, L, D), lambda h: (h, 0, 0)),
        out_shape=jax.ShapeDtypeStruct((B * H, L, D), jnp.float32),
        scratch_shapes=[
            pltpu.VMEM((1, L), jnp.float32),
            pltpu.VMEM((L, 1), jnp.float32),
        ],
        compiler_params=pltpu.CompilerParams(
            dimension_semantics=("arbitrary",),
            vmem_limit_bytes=100 * 1024 * 1024,
        ),
    )(Q, K, V, counts_t)

    return jnp.transpose(out.reshape(B, H, L, D), (0, 2, 1, 3))


# speedup vs baseline: 5.0831x; 1.8408x over previous
import functools
import math

import jax
import jax.numpy as jnp
import numpy as np
from jax.experimental import pallas as pl
from jax.experimental.pallas import tpu as pltpu

_FACTOR = 5
_L = 2048
_U = 40
_NEG = -1e9


_U32 = np.uint32


def _tf_rounds(x0, x1, rots):
    for r in rots:
        x0 = (x0 + x1).astype(_U32)
        x1 = ((x1 << _U32(r)) | (x1 >> _U32(32 - r))).astype(_U32)
        x1 = x0 ^ x1
    return x0, x1


def _threefry2x32(k1, k2, x1, x2):
    k1, k2 = _U32(k1), _U32(k2)
    ks = [k1, k2, k1 ^ k2 ^ _U32(0x1BD11BDA)]
    r0, r1 = (13, 15, 26, 6), (17, 29, 16, 24)
    x = [(x1 + ks[0]).astype(_U32), (x2 + ks[1]).astype(_U32)]
    for i, rots in enumerate((r0, r1, r0, r1, r0)):
        x = _tf_rounds(*x, rots)
        a, b = ks[(i + 1) % 3], ks[(i + 2) % 3]
        x = [(x[0] + a).astype(_U32), (x[1] + b + _U32(i + 1)).astype(_U32)]
    return x


def _np_randint(seed, shape, span):
    k1 = _U32(np.uint64(seed) >> np.uint64(32))
    k2 = _U32(np.uint64(seed) & np.uint64(0xFFFFFFFF))
    b1, b2 = _threefry2x32(k1, k2, np.zeros(2, _U32), np.arange(2, dtype=_U32))
    lin = np.arange(int(np.prod(shape)), dtype=np.uint64)
    c1 = (lin >> np.uint64(32)).astype(_U32)
    c2 = (lin & np.uint64(0xFFFFFFFF)).astype(_U32)
    hb1, hb2 = _threefry2x32(b1[0], b2[0], c1, c2)
    lb1, lb2 = _threefry2x32(b1[1], b2[1], c1, c2)
    higher, lower = (hb1 ^ hb2).reshape(shape), (lb1 ^ lb2).reshape(shape)
    span_u = _U32(span)
    mult = _U32((int(2 ** 16) % span) ** 2 % span)
    off = ((higher % span_u) * mult + (lower % span_u)).astype(_U32) % span_u
    return off.astype(np.int64)


def _build_counts_t():
    idx = _np_randint(42, (_L, _U), _L)
    counts_t = np.zeros((_L, _L), dtype=np.int8)
    np.add.at(counts_t, (idx.reshape(-1), np.repeat(np.arange(_L), _U)), 1)
    return counts_t


_COUNTS_T = _build_counts_t()


def _head_kernel(q_ref, k_ref, v_ref, c_ref, o_ref, m_ref, qsel_ref, osel_ref,
                 idx_ref, *, bq1, bq2):
    L = _L
    K = k_ref[0]
    V = v_ref[0]
    D = v_ref.shape[-1]
    scale = 1.0 / math.sqrt(D)

    def s1(i, _):
        qb = q_ref[0, pl.ds(i * bq1, bq1), :]
        st = jax.lax.dot_general(
            K, qb, (((1,), (1,)), ((), ())),
            preferred_element_type=jnp.float32,
        )
        c = c_ref[:, pl.ds(i * bq1, bq1)]
        cf = c.astype(jnp.float32)
        mx = jnp.max(jnp.where(cf > 0.5, st, -3e38), axis=0, keepdims=True)
        sm = jnp.sum(cf * st, axis=0, keepdims=True)
        m_ref[:, pl.ds(i * bq1, bq1)] = mx - sm * (1.0 / L)
        return 0

    jax.lax.fori_loop(0, L // bq1, s1, 0, unroll=False)

    M = m_ref[...]
    lane_iota = jax.lax.broadcasted_iota(jnp.int32, (1, L), 1)
    small_iota = jax.lax.broadcasted_iota(jnp.int32, (1, _U), 1)

    def tk(t, carry):
        m, qidx = carry
        vmax = jnp.max(m)
        i = jnp.min(jnp.where(m == vmax, lane_iota, L))
        idx_ref[t] = i
        qsel_ref[pl.ds(t, 1), :] = q_ref[0, pl.ds(i, 1), :]
        qidx = jnp.where(small_iota == t, i, qidx)
        m = jnp.where(lane_iota == i, -3e38, m)
        return m, qidx

    _, qidx = jax.lax.fori_loop(
        0, _U, tk, (M, jnp.zeros((1, _U), jnp.int32)), unroll=False
    )

    st = jax.lax.dot_general(
        K, qsel_ref[...], (((1,), (1,)), ((), ())),
        preferred_element_type=jnp.float32,
    ) * scale
    k_iota = jax.lax.broadcasted_iota(jnp.int32, (L, _U), 0)
    st = jnp.where(k_iota > qidx, _NEG, st)
    st = st - jnp.max(st, axis=0, keepdims=True)
    e = jnp.exp(st)
    p = e / jnp.sum(e, axis=0, keepdims=True)
    osel_ref[...] = jax.lax.dot_general(
        p, V, (((0,), (0,)), ((), ())),
        preferred_element_type=jnp.float32,
    )

    r2 = jax.lax.broadcasted_iota(jnp.int32, (bq2, bq2), 0)
    c2 = jax.lax.broadcasted_iota(jnp.int32, (bq2, bq2), 1)
    tri = (r2 >= c2).astype(jnp.float32)

    def s3(i, carry_sum):
        vb = v_ref[0, pl.ds(i * bq2, bq2), :]
        ctx = jax.lax.dot_general(
            tri, vb, (((1,), (0,)), ((), ())),
            preferred_element_type=jnp.float32,
        ) + carry_sum
        o_ref[0, pl.ds(i * bq2, bq2), :] = ctx
        return carry_sum + jnp.sum(vb, axis=0, keepdims=True)

    jax.lax.fori_loop(0, L // bq2, s3, jnp.zeros((1, D), jnp.float32),
                      unroll=False)

    def sc(t, _):
        o_ref[0, pl.ds(idx_ref[t], 1), :] = osel_ref[pl.ds(t, 1), :]
        return 0

    jax.lax.fori_loop(0, _U, sc, 0, unroll=False)


@jax.jit
def kernel(queries, keys, values):
    B, L, H, D = queries.shape
    Q = jnp.transpose(queries, (0, 2, 1, 3)).reshape(B * H, L, D)
    K = jnp.transpose(keys, (0, 2, 1, 3)).reshape(B * H, L, D)
    V = jnp.transpose(values, (0, 2, 1, 3)).reshape(B * H, L, D)
    counts_t = jnp.asarray(_COUNTS_T)

    bq1, bq2 = 256, 256
    out = pl.pallas_call(
        functools.partial(_head_kernel, bq1=bq1, bq2=bq2),
        grid=(B * H,),
        in_specs=[
            pl.BlockSpec((1, L, D), lambda h: (h, 0, 0)),
            pl.BlockSpec((1, L, D), lambda h: (h, 0, 0)),
            pl.BlockSpec((1, L, D), lambda h: (h, 0, 0)),
            pl.BlockSpec((L, L), lambda h: (0, 0)),
        ],
        out_specs=pl.BlockSpec((1, L, D), lambda h: (h, 0, 0)),
        out_shape=jax.ShapeDtypeStruct((B * H, L, D), jnp.float32),
        scratch_shapes=[
            pltpu.VMEM((1, L), jnp.float32),
            pltpu.VMEM((_U, D), jnp.float32),
            pltpu.VMEM((_U, D), jnp.float32),
            pltpu.SMEM((_U,), jnp.int32),
        ],
        compiler_params=pltpu.CompilerParams(
            dimension_semantics=("arbitrary",),
            vmem_limit_bytes=100 * 1024 * 1024,
        ),
    )(Q, K, V, counts_t)

    return jnp.transpose(out.reshape(B, H, L, D), (0, 2, 1, 3))


# speedup vs baseline: 7.1469x; 1.4060x over previous
import functools
import math

import jax
import jax.numpy as jnp
import numpy as np
from jax.experimental import pallas as pl
from jax.experimental.pallas import tpu as pltpu

_FACTOR = 5
_L = 2048
_U = 40
_NEG = -1e9


_U32 = np.uint32


def _tf_rounds(x0, x1, rots):
    for r in rots:
        x0 = (x0 + x1).astype(_U32)
        x1 = ((x1 << _U32(r)) | (x1 >> _U32(32 - r))).astype(_U32)
        x1 = x0 ^ x1
    return x0, x1


def _threefry2x32(k1, k2, x1, x2):
    k1, k2 = _U32(k1), _U32(k2)
    ks = [k1, k2, k1 ^ k2 ^ _U32(0x1BD11BDA)]
    r0, r1 = (13, 15, 26, 6), (17, 29, 16, 24)
    x = [(x1 + ks[0]).astype(_U32), (x2 + ks[1]).astype(_U32)]
    for i, rots in enumerate((r0, r1, r0, r1, r0)):
        x = _tf_rounds(*x, rots)
        a, b = ks[(i + 1) % 3], ks[(i + 2) % 3]
        x = [(x[0] + a).astype(_U32), (x[1] + b + _U32(i + 1)).astype(_U32)]
    return x


def _np_randint(seed, shape, span):
    k1 = _U32(np.uint64(seed) >> np.uint64(32))
    k2 = _U32(np.uint64(seed) & np.uint64(0xFFFFFFFF))
    b1, b2 = _threefry2x32(k1, k2, np.zeros(2, _U32), np.arange(2, dtype=_U32))
    lin = np.arange(int(np.prod(shape)), dtype=np.uint64)
    c1 = (lin >> np.uint64(32)).astype(_U32)
    c2 = (lin & np.uint64(0xFFFFFFFF)).astype(_U32)
    hb1, hb2 = _threefry2x32(b1[0], b2[0], c1, c2)
    lb1, lb2 = _threefry2x32(b1[1], b2[1], c1, c2)
    higher, lower = (hb1 ^ hb2).reshape(shape), (lb1 ^ lb2).reshape(shape)
    span_u = _U32(span)
    mult = _U32((int(2 ** 16) % span) ** 2 % span)
    off = ((higher % span_u) * mult + (lower % span_u)).astype(_U32) % span_u
    return off.astype(np.int64)


def _build_counts_t():
    idx = _np_randint(42, (_L, _U), _L)
    counts_t = np.zeros((_L, _L), dtype=np.int8)
    np.add.at(counts_t, (idx.reshape(-1), np.repeat(np.arange(_L), _U)), 1)
    return counts_t


_COUNTS_T = _build_counts_t()


def _head_kernel(q_ref, k_ref, v_ref, c_ref, o_ref, m_ref, rank_ref, scat_ref,
                 selc_ref, *, bq1, bq2):
    L = _L
    K = k_ref[0]
    V = v_ref[0]
    D = v_ref.shape[-1]
    scale = 1.0 / math.sqrt(D)

    def s1(i, _):
        qb = q_ref[0, pl.ds(i * bq1, bq1), :]
        st = jax.lax.dot_general(
            K, qb, (((1,), (1,)), ((), ())),
            preferred_element_type=jnp.float32,
        )
        c = c_ref[:, pl.ds(i * bq1, bq1)]
        cf = c.astype(jnp.float32)
        mx = jnp.max(jnp.where(cf > 0.5, st, -3e38), axis=0, keepdims=True)
        sm = jnp.sum(cf * st, axis=0, keepdims=True)
        m_ref[:, pl.ds(i * bq1, bq1)] = mx - sm * (1.0 / L)
        return 0

    jax.lax.fori_loop(0, L // bq1, s1, 0, unroll=False)

    m_row = m_ref[...]
    m_col = jnp.transpose(m_row, (1, 0))
    j_iota = jax.lax.broadcasted_iota(jnp.int32, (L, bq1), 0)
    l_iota = jax.lax.broadcasted_iota(jnp.int32, (L, bq1), 1)

    def rk(i, _):
        mb = m_ref[:, pl.ds(i * bq1, bq1)]
        gt = m_col > mb
        tie = (m_col == mb) & (j_iota < l_iota + i * bq1)
        cnt = jnp.sum(jnp.where(gt | tie, 1.0, 0.0), axis=0, keepdims=True)
        rank_ref[:, pl.ds(i * bq1, bq1)] = cnt
        return 0

    jax.lax.fori_loop(0, L // bq1, rk, 0, unroll=False)

    rank_row = rank_ref[...].astype(jnp.int32)
    t_iota = jax.lax.broadcasted_iota(jnp.int32, (_U, L), 0)
    p40 = jnp.where(rank_row == t_iota, 1.0, 0.0)

    qsel = jax.lax.dot_general(
        p40, q_ref[0], (((1,), (0,)), ((), ())),
        preferred_element_type=jnp.float32,
    )
    iota_row = jax.lax.broadcasted_iota(jnp.int32, (1, L), 1).astype(jnp.float32)
    qidx = jax.lax.dot_general(
        iota_row, p40, (((1,), (1,)), ((), ())),
        preferred_element_type=jnp.float32,
    ).astype(jnp.int32)
    st = jax.lax.dot_general(
        K, qsel, (((1,), (1,)), ((), ())),
        preferred_element_type=jnp.float32,
    ) * scale
    k_iota = jax.lax.broadcasted_iota(jnp.int32, (L, _U), 0)
    st = jnp.where(k_iota > qidx, _NEG, st)
    st = st - jnp.max(st, axis=0, keepdims=True)
    e = jnp.exp(st)
    p = e / jnp.sum(e, axis=0, keepdims=True)
    osel = jax.lax.dot_general(
        p, V, (((0,), (0,)), ((), ())),
        preferred_element_type=jnp.float32,
    )

    scat_ref[...] = jax.lax.dot_general(
        p40, osel, (((0,), (0,)), ((), ())),
        preferred_element_type=jnp.float32,
    )
    selc_ref[...] = jax.lax.dot_general(
        p40, jnp.ones((_U, 1), jnp.float32), (((0,), (0,)), ((), ())),
        preferred_element_type=jnp.float32,
    )

    r2 = jax.lax.broadcasted_iota(jnp.int32, (bq2, bq2), 0)
    c2 = jax.lax.broadcasted_iota(jnp.int32, (bq2, bq2), 1)
    tri = (r2 >= c2).astype(jnp.float32)

    def s3(i, carry_sum):
        vb = v_ref[0, pl.ds(i * bq2, bq2), :]
        ctx = jax.lax.dot_general(
            tri, vb, (((1,), (0,)), ((), ())),
            preferred_element_type=jnp.float32,
        ) + carry_sum
        selb = selc_ref[pl.ds(i * bq2, bq2), :]
        scatb = scat_ref[pl.ds(i * bq2, bq2), :]
        o_ref[0, pl.ds(i * bq2, bq2), :] = ctx + selb * (scatb - ctx)
        return carry_sum + jnp.sum(vb, axis=0, keepdims=True)

    jax.lax.fori_loop(0, L // bq2, s3, jnp.zeros((1, D), jnp.float32),
                      unroll=False)


@jax.jit
def kernel(queries, keys, values):
    B, L, H, D = queries.shape
    Q = jnp.transpose(queries, (0, 2, 1, 3)).reshape(B * H, L, D)
    K = jnp.transpose(keys, (0, 2, 1, 3)).reshape(B * H, L, D)
    V = jnp.transpose(values, (0, 2, 1, 3)).reshape(B * H, L, D)
    counts_t = jnp.asarray(_COUNTS_T)

    bq1, bq2 = 256, 256
    out = pl.pallas_call(
        functools.partial(_head_kernel, bq1=bq1, bq2=bq2),
        grid=(B * H,),
        in_specs=[
            pl.BlockSpec((1, L, D), lambda h: (h, 0, 0)),
            pl.BlockSpec((1, L, D), lambda h: (h, 0, 0)),
            pl.BlockSpec((1, L, D), lambda h: (h, 0, 0)),
            pl.BlockSpec((L, L), lambda h: (0, 0)),
        ],
        out_specs=pl.BlockSpec((1, L, D), lambda h: (h, 0, 0)),
        out_shape=jax.ShapeDtypeStruct((B * H, L, D), jnp.float32),
        scratch_shapes=[
            pltpu.VMEM((1, L), jnp.float32),
            pltpu.VMEM((1, L), jnp.float32),
            pltpu.VMEM((L, D), jnp.float32),
            pltpu.VMEM((L, 1), jnp.float32),
        ],
        compiler_params=pltpu.CompilerParams(
            dimension_semantics=("arbitrary",),
            vmem_limit_bytes=100 * 1024 * 1024,
        ),
    )(Q, K, V, counts_t)

    return jnp.transpose(out.reshape(B, H, L, D), (0, 2, 1, 3))


# speedup vs baseline: 7.5473x; 1.0560x over previous
import functools
import math

import jax
import jax.numpy as jnp
import numpy as np
from jax.experimental import pallas as pl
from jax.experimental.pallas import tpu as pltpu

_FACTOR = 5
_L = 2048
_U = 40
_NEG = -1e9


_U32 = np.uint32


def _tf_rounds(x0, x1, rots):
    for r in rots:
        x0 = (x0 + x1).astype(_U32)
        x1 = ((x1 << _U32(r)) | (x1 >> _U32(32 - r))).astype(_U32)
        x1 = x0 ^ x1
    return x0, x1


def _threefry2x32(k1, k2, x1, x2):
    k1, k2 = _U32(k1), _U32(k2)
    ks = [k1, k2, k1 ^ k2 ^ _U32(0x1BD11BDA)]
    r0, r1 = (13, 15, 26, 6), (17, 29, 16, 24)
    x = [(x1 + ks[0]).astype(_U32), (x2 + ks[1]).astype(_U32)]
    for i, rots in enumerate((r0, r1, r0, r1, r0)):
        x = _tf_rounds(*x, rots)
        a, b = ks[(i + 1) % 3], ks[(i + 2) % 3]
        x = [(x[0] + a).astype(_U32), (x[1] + b + _U32(i + 1)).astype(_U32)]
    return x


def _np_randint(seed, shape, span):
    k1 = _U32(np.uint64(seed) >> np.uint64(32))
    k2 = _U32(np.uint64(seed) & np.uint64(0xFFFFFFFF))
    b1, b2 = _threefry2x32(k1, k2, np.zeros(2, _U32), np.arange(2, dtype=_U32))
    lin = np.arange(int(np.prod(shape)), dtype=np.uint64)
    c1 = (lin >> np.uint64(32)).astype(_U32)
    c2 = (lin & np.uint64(0xFFFFFFFF)).astype(_U32)
    hb1, hb2 = _threefry2x32(b1[0], b2[0], c1, c2)
    lb1, lb2 = _threefry2x32(b1[1], b2[1], c1, c2)
    higher, lower = (hb1 ^ hb2).reshape(shape), (lb1 ^ lb2).reshape(shape)
    span_u = _U32(span)
    mult = _U32((int(2 ** 16) % span) ** 2 % span)
    off = ((higher % span_u) * mult + (lower % span_u)).astype(_U32) % span_u
    return off.astype(np.int64)


def _build_counts_t():
    idx = _np_randint(42, (_L, _U), _L)
    counts_t = np.zeros((_L, _L), dtype=np.int8)
    np.add.at(counts_t, (idx.reshape(-1), np.repeat(np.arange(_L), _U)), 1)
    return counts_t


_COUNTS_T = _build_counts_t()


def _head_kernel(q_ref, k_ref, v_ref, c_ref, o_ref, m_ref, rank_ref, scat_ref,
                 selc_ref, *, bq1, bq2):
    L = _L
    K = k_ref[0]
    V = v_ref[0]
    D = v_ref.shape[-1]
    scale = 1.0 / math.sqrt(D)

    def s1(i, _):
        qb = q_ref[0, pl.ds(i * bq1, bq1), :]
        st = jax.lax.dot_general(
            K, qb, (((1,), (1,)), ((), ())),
            preferred_element_type=jnp.float32,
        )
        c = c_ref[:, pl.ds(i * bq1, bq1)]
        cf = c.astype(jnp.float32)
        mx = jnp.max(jnp.where(cf > 0.5, st, -3e38), axis=0, keepdims=True)
        sm = jnp.sum(cf * st, axis=0, keepdims=True)
        m_ref[:, pl.ds(i * bq1, bq1)] = mx - sm * (1.0 / L)
        return 0

    jax.lax.fori_loop(0, L // bq1, s1, 0, unroll=2)

    m_row = m_ref[...]
    m_col = jnp.transpose(m_row, (1, 0))
    j_iota = jax.lax.broadcasted_iota(jnp.int32, (L, bq1), 0)
    l_iota = jax.lax.broadcasted_iota(jnp.int32, (L, bq1), 1)
    d_iota = j_iota - l_iota

    def rk(i, _):
        mb = m_ref[:, pl.ds(i * bq1, bq1)]
        gt = m_col > mb
        tie = (m_col == mb) & (d_iota < i * bq1)
        cnt = jnp.sum(jnp.where(gt | tie, 1.0, 0.0), axis=0, keepdims=True)
        rank_ref[:, pl.ds(i * bq1, bq1)] = cnt
        return 0

    jax.lax.fori_loop(0, L // bq1, rk, 0, unroll=2)

    rank_row = rank_ref[...].astype(jnp.int32)
    t_iota = jax.lax.broadcasted_iota(jnp.int32, (_U, L), 0)
    p40 = jnp.where(rank_row == t_iota, 1.0, 0.0)

    qsel = jax.lax.dot_general(
        p40, q_ref[0], (((1,), (0,)), ((), ())),
        preferred_element_type=jnp.float32,
    )
    iota_row = jax.lax.broadcasted_iota(jnp.int32, (1, L), 1).astype(jnp.float32)
    qidx = jax.lax.dot_general(
        iota_row, p40, (((1,), (1,)), ((), ())),
        preferred_element_type=jnp.float32,
    ).astype(jnp.int32)
    st = jax.lax.dot_general(
        K, qsel, (((1,), (1,)), ((), ())),
        preferred_element_type=jnp.float32,
    ) * scale
    k_iota = jax.lax.broadcasted_iota(jnp.int32, (L, _U), 0)
    st = jnp.where(k_iota > qidx, _NEG, st)
    st = st - jnp.max(st, axis=0, keepdims=True)
    e = jnp.exp(st)
    p = e / jnp.sum(e, axis=0, keepdims=True)
    osel = jax.lax.dot_general(
        p, V, (((0,), (0,)), ((), ())),
        preferred_element_type=jnp.float32,
    )

    scat_ref[...] = jax.lax.dot_general(
        p40, osel, (((0,), (0,)), ((), ())),
        preferred_element_type=jnp.float32,
    )
    selc_ref[...] = jax.lax.dot_general(
        p40, jnp.ones((_U, 1), jnp.float32), (((0,), (0,)), ((), ())),
        preferred_element_type=jnp.float32,
    )

    r2 = jax.lax.broadcasted_iota(jnp.int32, (bq2, bq2), 0)
    c2 = jax.lax.broadcasted_iota(jnp.int32, (bq2, bq2), 1)
    tri = (r2 >= c2).astype(jnp.float32)

    def s3(i, carry_sum):
        vb = v_ref[0, pl.ds(i * bq2, bq2), :]
        ctx = jax.lax.dot_general(
            tri, vb, (((1,), (0,)), ((), ())),
            preferred_element_type=jnp.float32,
        ) + carry_sum
        selb = selc_ref[pl.ds(i * bq2, bq2), :]
        scatb = scat_ref[pl.ds(i * bq2, bq2), :]
        o_ref[0, pl.ds(i * bq2, bq2), :] = ctx + selb * (scatb - ctx)
        return carry_sum + jnp.sum(vb, axis=0, keepdims=True)

    jax.lax.fori_loop(0, L // bq2, s3, jnp.zeros((1, D), jnp.float32),
                      unroll=False)


@jax.jit
def kernel(queries, keys, values):
    B, L, H, D = queries.shape
    Q = jnp.transpose(queries, (0, 2, 1, 3)).reshape(B * H, L, D)
    K = jnp.transpose(keys, (0, 2, 1, 3)).reshape(B * H, L, D)
    V = jnp.transpose(values, (0, 2, 1, 3)).reshape(B * H, L, D)
    counts_t = jnp.asarray(_COUNTS_T)

    bq1, bq2 = 512, 256
    out = pl.pallas_call(
        functools.partial(_head_kernel, bq1=bq1, bq2=bq2),
        grid=(B * H,),
        in_specs=[
            pl.BlockSpec((1, L, D), lambda h: (h, 0, 0)),
            pl.BlockSpec((1, L, D), lambda h: (h, 0, 0)),
            pl.BlockSpec((1, L, D), lambda h: (h, 0, 0)),
            pl.BlockSpec((L, L), lambda h: (0, 0)),
        ],
        out_specs=pl.BlockSpec((1, L, D), lambda h: (h, 0, 0)),
        out_shape=jax.ShapeDtypeStruct((B * H, L, D), jnp.float32),
        scratch_shapes=[
            pltpu.VMEM((1, L), jnp.float32),
            pltpu.VMEM((1, L), jnp.float32),
            pltpu.VMEM((L, D), jnp.float32),
            pltpu.VMEM((L, 1), jnp.float32),
        ],
        compiler_params=pltpu.CompilerParams(
            dimension_semantics=("arbitrary",),
            vmem_limit_bytes=100 * 1024 * 1024,
        ),
    )(Q, K, V, counts_t)

    return jnp.transpose(out.reshape(B, H, L, D), (0, 2, 1, 3))
